# baseline (speedup 1.0000x reference)
import jax
import jax.numpy as jnp
from jax import lax
from jax.experimental import pallas as pl
from jax.experimental.pallas import tpu as pltpu
from jax.experimental.pallas import tpu_sc as plsc

MAX_TOKENS = 32768
MAX_SEQS = 128
NEW = 4096
NC = 1
NS = 16
NW = NC * NS
CHUNK = MAX_TOKENS // NW
VECS = CHUNK // 16
NSTAGE = CHUNK + 16
FW = MAX_SEQS // 4
SCAL = 32 + 2 * FW

_mesh = plsc.VectorSubcoreMesh(core_axis_name="c", subcore_axis_name="s",
                               num_cores=NC)


def _body(tok_hbm, sid_hbm, lp_hbm, scal_hbm,
          ntok_hbm, nsid_hbm, nlp_hbm,
          out_tok, out_sid, out_lp, out_fin,
          scal_v, oldt, olds, oldl, newt, news, newl, finv,
          sem_scal, sem_in, sem_new, sem_out, sem_fin):
    wid = lax.axis_index("s") * NC + lax.axis_index("c")
    base = wid * CHUNK

    pltpu.async_copy(scal_hbm, scal_v, sem_scal).wait()
    t_vec = scal_v[pl.ds(0, 16)]
    n_vec = jnp.minimum(scal_v[pl.ds(16, 16)], NEW)
    end_vec = t_vec + n_vec
    t_s = t_vec[0]
    n_s = n_vec[0]

    jlo = jnp.clip((t_s - base) >> 4, 0, VECS)
    jhi = jnp.clip((t_s + n_s - base + 15) >> 4, 0, VECS)

    @pl.when(wid == NW - 1)
    def _fin():
        finv[pl.ds(0, 16)] = scal_v[pl.ds(32, 16)] | scal_v[pl.ds(64, 16)]
        finv[pl.ds(16, 16)] = scal_v[pl.ds(48, 16)] | scal_v[pl.ds(80, 16)]
        pltpu.async_copy(finv, out_fin, sem_fin).wait()

    full = ((t_s <= base) & (t_s + n_s >= base + CHUNK) & ((t_s & 7) == 0))
    none = jhi <= jlo

    @pl.when(none)
    def _copy_only():
        d1 = pltpu.async_copy(tok_hbm.at[pl.ds(base, CHUNK)],
                              out_tok.at[pl.ds(base, CHUNK)], sem_out)
        d2 = pltpu.async_copy(sid_hbm.at[pl.ds(base, CHUNK)],
                              out_sid.at[pl.ds(base, CHUNK)], sem_out)
        d3 = pltpu.async_copy(lp_hbm.at[pl.ds(base, CHUNK)],
                              out_lp.at[pl.ds(base, CHUNK)], sem_out)
        d1.wait()
        d2.wait()
        d3.wait()

    @pl.when(full)
    def _full_cover():
        src = pl.multiple_of(base - t_s, 8)
        d1 = pltpu.async_copy(ntok_hbm.at[pl.ds(src, CHUNK)], oldt, sem_in)
        d2 = pltpu.async_copy(nsid_hbm.at[pl.ds(src, CHUNK)], olds, sem_in)
        d3 = pltpu.async_copy(nlp_hbm.at[pl.ds(src, CHUNK)], oldl, sem_in)
        d1.wait()
        d2.wait()
        d3.wait()
        o1 = pltpu.async_copy(oldt, out_tok.at[pl.ds(base, CHUNK)], sem_out)
        o2 = pltpu.async_copy(olds, out_sid.at[pl.ds(base, CHUNK)], sem_out)
        o3 = pltpu.async_copy(oldl, out_lp.at[pl.ds(base, CHUNK)], sem_out)
        o1.wait()
        o2.wait()
        o3.wait()

    @pl.when(jnp.logical_not(none | full))
    def _patch():
        di1 = pltpu.async_copy(tok_hbm.at[pl.ds(base, CHUNK)], oldt, sem_in)
        di2 = pltpu.async_copy(sid_hbm.at[pl.ds(base, CHUNK)], olds, sem_in)
        di3 = pltpu.async_copy(lp_hbm.at[pl.ds(base, CHUNK)], oldl, sem_in)
        start = pl.multiple_of(
            jnp.clip((base - t_s) & ~15, 0, NEW - NSTAGE), 16)
        dn1 = pltpu.async_copy(ntok_hbm.at[pl.ds(start, NSTAGE)], newt, sem_new)
        dn2 = pltpu.async_copy(nsid_hbm.at[pl.ds(start, NSTAGE)], news, sem_new)
        dn3 = pltpu.async_copy(nlp_hbm.at[pl.ds(start, NSTAGE)], newl, sem_new)
        di1.wait()
        di2.wait()
        di3.wait()
        dn1.wait()
        dn2.wait()
        dn3.wait()

        iota = lax.iota(jnp.int32, 16)
        shift_vec = t_vec + start

        def jbody(j, carry):
            i0 = j * 16
            lane = i0 + iota
            idx = base + lane
            m = (idx >= t_vec) & (idx < end_vec)
            off = jnp.clip(idx - shift_vec, 0, NSTAGE - 1)
            vt = plsc.load_gather(newt, [off])
            vs = plsc.load_gather(news, [off])
            vl = plsc.load_gather(newl, [off])
            plsc.store_scatter(oldt, [lane], vt, mask=m)
            plsc.store_scatter(olds, [lane], vs, mask=m)
            plsc.store_scatter(oldl, [lane], vl, mask=m)
            return carry

        lax.fori_loop(jlo, jhi, jbody, 0)

        do1 = pltpu.async_copy(oldt, out_tok.at[pl.ds(base, CHUNK)], sem_out)
        do2 = pltpu.async_copy(olds, out_sid.at[pl.ds(base, CHUNK)], sem_out)
        do3 = pltpu.async_copy(oldl, out_lp.at[pl.ds(base, CHUNK)], sem_out)
        do1.wait()
        do2.wait()
        do3.wait()


_sc_update = pl.kernel(
    _body,
    out_type=(
        jax.ShapeDtypeStruct((MAX_TOKENS,), jnp.int32),
        jax.ShapeDtypeStruct((MAX_TOKENS,), jnp.int32),
        jax.ShapeDtypeStruct((MAX_TOKENS,), jnp.float32),
        jax.ShapeDtypeStruct((FW,), jnp.int32),
    ),
    mesh=_mesh,
    scratch_types=[
        pltpu.VMEM((SCAL,), jnp.int32),
        pltpu.VMEM((CHUNK,), jnp.int32),
        pltpu.VMEM((CHUNK,), jnp.int32),
        pltpu.VMEM((CHUNK,), jnp.float32),
        pltpu.VMEM((NSTAGE,), jnp.int32),
        pltpu.VMEM((NSTAGE,), jnp.int32),
        pltpu.VMEM((NSTAGE,), jnp.float32),
        pltpu.VMEM((2 * 16,), jnp.int32),
        pltpu.SemaphoreType.DMA,
        pltpu.SemaphoreType.DMA,
        pltpu.SemaphoreType.DMA,
        pltpu.SemaphoreType.DMA,
        pltpu.SemaphoreType.DMA,
    ],
    compiler_params=pltpu.CompilerParams(needs_layout_passes=False),
)


def kernel(tokens_buf, slot_ids_buf, logprobs_buf, num_tokens, finished,
           new_tokens, new_slot_ids, new_logprobs, num_new_tokens,
           finished_snapshot):
    t = jnp.asarray(num_tokens, jnp.int32)
    n = jnp.asarray(num_new_tokens, jnp.int32)
    fin_w = lax.bitcast_convert_type(
        finished.astype(jnp.uint8).reshape(FW, 4), jnp.int32)
    snap_w = lax.bitcast_convert_type(
        finished_snapshot.astype(jnp.uint8).reshape(FW, 4), jnp.int32)
    scal = jnp.concatenate(
        [jnp.broadcast_to(t, (16,)), jnp.broadcast_to(n, (16,)),
         fin_w, snap_w])
    out_tok, out_sid, out_lp, out_fin = _sc_update(
        tokens_buf, slot_ids_buf, logprobs_buf, scal,
        new_tokens, new_slot_ids, new_logprobs)
    fin_bool = lax.bitcast_convert_type(out_fin, jnp.uint8).reshape(MAX_SEQS)
    return (out_tok, out_sid, out_lp, t + n, fin_bool.astype(jnp.bool_))


# speedup vs baseline: 1.3762x; 1.3762x over previous
import jax
import jax.numpy as jnp
from jax import lax
from jax.experimental import pallas as pl
from jax.experimental.pallas import tpu as pltpu
from jax.experimental.pallas import tpu_sc as plsc

MAX_TOKENS = 32768
MAX_SEQS = 128
NEW = 4096
NC = 1
NS = 16
NW = NC * NS
CHUNK = MAX_TOKENS // NW
VECS = CHUNK // 16
NSTAGE = CHUNK + 16
FW = MAX_SEQS // 4
SCAL = 32 + 2 * FW

_mesh = plsc.VectorSubcoreMesh(core_axis_name="c", subcore_axis_name="s",
                               num_cores=NC)


def _body(tok_hbm, sid_hbm, lp_hbm, scal_hbm,
          ntok_hbm, nsid_hbm, nlp_hbm,
          out_tok, out_sid, out_lp, out_fin,
          scal_v, oldt, olds, oldl, newt, news, newl, finv,
          sem_scal, sem_in, sem_new, sem_out, sem_fin):
    wid = lax.axis_index("s") * NC + lax.axis_index("c")
    base = wid * CHUNK

    pltpu.async_copy(scal_hbm, scal_v, sem_scal).wait()
    t_vec = scal_v[pl.ds(0, 16)]
    n_vec = jnp.minimum(scal_v[pl.ds(16, 16)], NEW)
    end_vec = t_vec + n_vec
    t_s = t_vec[0]
    n_s = n_vec[0]

    jlo = jnp.clip((t_s - base) >> 4, 0, VECS)
    jhi = jnp.clip((t_s + n_s - base + 15) >> 4, 0, VECS)

    @pl.when(wid == NW - 1)
    def _fin():
        finv[pl.ds(0, 16)] = scal_v[pl.ds(32, 16)] | scal_v[pl.ds(64, 16)]
        finv[pl.ds(16, 16)] = scal_v[pl.ds(48, 16)] | scal_v[pl.ds(80, 16)]
        pltpu.async_copy(finv, out_fin, sem_fin).wait()

    full = ((t_s <= base) & (t_s + n_s >= base + CHUNK) & ((t_s & 7) == 0))
    partial = (jlo < jhi) & jnp.logical_not(full)

    @pl.when(jnp.logical_not(full))
    def _stage_old():
        pltpu.async_copy(tok_hbm.at[pl.ds(base, CHUNK)], oldt, sem_in)
        pltpu.async_copy(sid_hbm.at[pl.ds(base, CHUNK)], olds, sem_in)
        pltpu.async_copy(lp_hbm.at[pl.ds(base, CHUNK)], oldl, sem_in)

    @pl.when(full)
    def _stage_full():
        src = pl.multiple_of(base - t_s, 8)
        pltpu.async_copy(ntok_hbm.at[pl.ds(src, CHUNK)], oldt, sem_in)
        pltpu.async_copy(nsid_hbm.at[pl.ds(src, CHUNK)], olds, sem_in)
        pltpu.async_copy(nlp_hbm.at[pl.ds(src, CHUNK)], oldl, sem_in)

    start = pl.multiple_of(jnp.clip((base - t_s) & ~15, 0, NEW - NSTAGE), 16)

    @pl.when(partial)
    def _stage_new():
        pltpu.async_copy(ntok_hbm.at[pl.ds(start, NSTAGE)], newt, sem_new)
        pltpu.async_copy(nsid_hbm.at[pl.ds(start, NSTAGE)], news, sem_new)
        pltpu.async_copy(nlp_hbm.at[pl.ds(start, NSTAGE)], newl, sem_new)

    pltpu.make_async_copy(tok_hbm.at[pl.ds(base, CHUNK)], oldt, sem_in).wait()
    pltpu.make_async_copy(sid_hbm.at[pl.ds(base, CHUNK)], olds, sem_in).wait()
    pltpu.make_async_copy(lp_hbm.at[pl.ds(base, CHUNK)], oldl, sem_in).wait()

    @pl.when(partial)
    def _patch():
        pltpu.make_async_copy(ntok_hbm.at[pl.ds(start, NSTAGE)], newt,
                              sem_new).wait()
        pltpu.make_async_copy(nsid_hbm.at[pl.ds(start, NSTAGE)], news,
                              sem_new).wait()
        pltpu.make_async_copy(nlp_hbm.at[pl.ds(start, NSTAGE)], newl,
                              sem_new).wait()

        iota = lax.iota(jnp.int32, 16)
        shift_vec = t_vec + start

        def jbody(j, carry):
            i0 = j * 16
            lane = i0 + iota
            idx = base + lane
            m = (idx >= t_vec) & (idx < end_vec)
            off = jnp.clip(idx - shift_vec, 0, NSTAGE - 1)
            vt = plsc.load_gather(newt, [off])
            vs = plsc.load_gather(news, [off])
            vl = plsc.load_gather(newl, [off])
            plsc.store_scatter(oldt, [lane], vt, mask=m)
            plsc.store_scatter(olds, [lane], vs, mask=m)
            plsc.store_scatter(oldl, [lane], vl, mask=m)
            return carry

        lax.fori_loop(jlo, jhi, jbody, 0)

    do1 = pltpu.async_copy(oldt, out_tok.at[pl.ds(base, CHUNK)], sem_out)
    do2 = pltpu.async_copy(olds, out_sid.at[pl.ds(base, CHUNK)], sem_out)
    do3 = pltpu.async_copy(oldl, out_lp.at[pl.ds(base, CHUNK)], sem_out)
    do1.wait()
    do2.wait()
    do3.wait()


_sc_update = pl.kernel(
    _body,
    out_type=(
        jax.ShapeDtypeStruct((MAX_TOKENS,), jnp.int32),
        jax.ShapeDtypeStruct((MAX_TOKENS,), jnp.int32),
        jax.ShapeDtypeStruct((MAX_TOKENS,), jnp.float32),
        jax.ShapeDtypeStruct((FW,), jnp.int32),
    ),
    mesh=_mesh,
    scratch_types=[
        pltpu.VMEM((SCAL,), jnp.int32),
        pltpu.VMEM((CHUNK,), jnp.int32),
        pltpu.VMEM((CHUNK,), jnp.int32),
        pltpu.VMEM((CHUNK,), jnp.float32),
        pltpu.VMEM((NSTAGE,), jnp.int32),
        pltpu.VMEM((NSTAGE,), jnp.int32),
        pltpu.VMEM((NSTAGE,), jnp.float32),
        pltpu.VMEM((2 * 16,), jnp.int32),
        pltpu.SemaphoreType.DMA,
        pltpu.SemaphoreType.DMA,
        pltpu.SemaphoreType.DMA,
        pltpu.SemaphoreType.DMA,
        pltpu.SemaphoreType.DMA,
    ],
    compiler_params=pltpu.CompilerParams(needs_layout_passes=False),
)


def kernel(tokens_buf, slot_ids_buf, logprobs_buf, num_tokens, finished,
           new_tokens, new_slot_ids, new_logprobs, num_new_tokens,
           finished_snapshot):
    t = jnp.asarray(num_tokens, jnp.int32)
    n = jnp.asarray(num_new_tokens, jnp.int32)
    fin_w = lax.bitcast_convert_type(
        finished.astype(jnp.uint8).reshape(FW, 4), jnp.int32)
    snap_w = lax.bitcast_convert_type(
        finished_snapshot.astype(jnp.uint8).reshape(FW, 4), jnp.int32)
    scal = jnp.concatenate(
        [jnp.broadcast_to(t, (16,)), jnp.broadcast_to(n, (16,)),
         fin_w, snap_w])
    out_tok, out_sid, out_lp, out_fin = _sc_update(
        tokens_buf, slot_ids_buf, logprobs_buf, scal,
        new_tokens, new_slot_ids, new_logprobs)
    fin_bool = lax.bitcast_convert_type(out_fin, jnp.uint8).reshape(MAX_SEQS)
    return (out_tok, out_sid, out_lp, t + n, fin_bool.astype(jnp.bool_))


# speedup vs baseline: 1.3778x; 1.0011x over previous
import jax
import jax.numpy as jnp
from jax import lax
from jax.experimental import pallas as pl
from jax.experimental.pallas import tpu as pltpu
from jax.experimental.pallas import tpu_sc as plsc

MAX_TOKENS = 32768
MAX_SEQS = 128
NEW = 4096
NC = 1
NS = 16
NW = NC * NS
CHUNK = MAX_TOKENS // NW
VECS = CHUNK // 16
NSTAGE = CHUNK + 16
FW = MAX_SEQS // 4
SCAL = 32 + 2 * FW

_mesh = plsc.VectorSubcoreMesh(core_axis_name="c", subcore_axis_name="s",
                               num_cores=NC)


def _body(tok_hbm, sid_hbm, lp_hbm, scal_hbm,
          ntok_hbm, nsid_hbm, nlp_hbm,
          out_tok, out_sid, out_lp, out_fin,
          scal_v, oldt, olds, oldl, newt, news, newl, finv,
          sem_scal, sem_in, sem_new, sem_out, sem_fin):
    wid = lax.axis_index("s") * NC + lax.axis_index("c")
    base = wid * CHUNK

    pltpu.async_copy(tok_hbm.at[pl.ds(base, CHUNK)], oldt, sem_in)
    pltpu.async_copy(sid_hbm.at[pl.ds(base, CHUNK)], olds, sem_in)
    pltpu.async_copy(lp_hbm.at[pl.ds(base, CHUNK)], oldl, sem_in)

    pltpu.async_copy(scal_hbm, scal_v, sem_scal).wait()
    t_vec = scal_v[pl.ds(0, 16)]
    n_vec = jnp.minimum(scal_v[pl.ds(16, 16)], NEW)
    end_vec = t_vec + n_vec
    t_s = t_vec[0]
    n_s = n_vec[0]

    jlo = jnp.clip((t_s - base) >> 4, 0, VECS)
    jhi = jnp.clip((t_s + n_s - base + 15) >> 4, 0, VECS)

    @pl.when(wid == NW - 1)
    def _fin():
        finv[pl.ds(0, 16)] = scal_v[pl.ds(32, 16)] | scal_v[pl.ds(64, 16)]
        finv[pl.ds(16, 16)] = scal_v[pl.ds(48, 16)] | scal_v[pl.ds(80, 16)]
        pltpu.async_copy(finv, out_fin, sem_fin).wait()

    full = ((t_s <= base) & (t_s + n_s >= base + CHUNK) & ((t_s & 7) == 0))
    partial = (jlo < jhi) & jnp.logical_not(full)

    start = pl.multiple_of(jnp.clip((base - t_s) & ~15, 0, NEW - NSTAGE), 16)

    @pl.when(full)
    def _stage_full():
        src = pl.multiple_of(base - t_s, 8)
        pltpu.async_copy(ntok_hbm.at[pl.ds(src, CHUNK)],
                         newt.at[pl.ds(0, CHUNK)], sem_new)
        pltpu.async_copy(nsid_hbm.at[pl.ds(src, CHUNK)],
                         news.at[pl.ds(0, CHUNK)], sem_new)
        pltpu.async_copy(nlp_hbm.at[pl.ds(src, CHUNK)],
                         newl.at[pl.ds(0, CHUNK)], sem_new)

    @pl.when(partial)
    def _stage_new():
        pltpu.async_copy(ntok_hbm.at[pl.ds(start, NSTAGE)], newt, sem_new)
        pltpu.async_copy(nsid_hbm.at[pl.ds(start, NSTAGE)], news, sem_new)
        pltpu.async_copy(nlp_hbm.at[pl.ds(start, NSTAGE)], newl, sem_new)

    pltpu.make_async_copy(tok_hbm.at[pl.ds(base, CHUNK)], oldt, sem_in).wait()
    pltpu.make_async_copy(sid_hbm.at[pl.ds(base, CHUNK)], olds, sem_in).wait()
    pltpu.make_async_copy(lp_hbm.at[pl.ds(base, CHUNK)], oldl, sem_in).wait()

    @pl.when(partial)
    def _patch():
        pltpu.make_async_copy(ntok_hbm.at[pl.ds(start, NSTAGE)], newt,
                              sem_new).wait()
        pltpu.make_async_copy(nsid_hbm.at[pl.ds(start, NSTAGE)], news,
                              sem_new).wait()
        pltpu.make_async_copy(nlp_hbm.at[pl.ds(start, NSTAGE)], newl,
                              sem_new).wait()

        iota = lax.iota(jnp.int32, 16)
        shift_vec = t_vec + start

        def jbody(j, carry):
            i0 = j * 16
            lane = i0 + iota
            idx = base + lane
            m = (idx >= t_vec) & (idx < end_vec)
            off = jnp.clip(idx - shift_vec, 0, NSTAGE - 1)
            vt = plsc.load_gather(newt, [off])
            vs = plsc.load_gather(news, [off])
            vl = plsc.load_gather(newl, [off])
            plsc.store_scatter(oldt, [lane], vt, mask=m)
            plsc.store_scatter(olds, [lane], vs, mask=m)
            plsc.store_scatter(oldl, [lane], vl, mask=m)
            return carry

        lax.fori_loop(jlo, jhi, jbody, 0)

    @pl.when(full)
    def _out_full():
        pltpu.make_async_copy(ntok_hbm.at[pl.ds(start, CHUNK)],
                              newt.at[pl.ds(0, CHUNK)], sem_new).wait()
        pltpu.make_async_copy(nsid_hbm.at[pl.ds(start, CHUNK)],
                              news.at[pl.ds(0, CHUNK)], sem_new).wait()
        pltpu.make_async_copy(nlp_hbm.at[pl.ds(start, CHUNK)],
                              newl.at[pl.ds(0, CHUNK)], sem_new).wait()
        pltpu.async_copy(newt.at[pl.ds(0, CHUNK)],
                         out_tok.at[pl.ds(base, CHUNK)], sem_out)
        pltpu.async_copy(news.at[pl.ds(0, CHUNK)],
                         out_sid.at[pl.ds(base, CHUNK)], sem_out)
        pltpu.async_copy(newl.at[pl.ds(0, CHUNK)],
                         out_lp.at[pl.ds(base, CHUNK)], sem_out)

    @pl.when(jnp.logical_not(full))
    def _out_old():
        pltpu.async_copy(oldt, out_tok.at[pl.ds(base, CHUNK)], sem_out)
        pltpu.async_copy(olds, out_sid.at[pl.ds(base, CHUNK)], sem_out)
        pltpu.async_copy(oldl, out_lp.at[pl.ds(base, CHUNK)], sem_out)

    pltpu.make_async_copy(oldt, out_tok.at[pl.ds(base, CHUNK)], sem_out).wait()
    pltpu.make_async_copy(olds, out_sid.at[pl.ds(base, CHUNK)], sem_out).wait()
    pltpu.make_async_copy(oldl, out_lp.at[pl.ds(base, CHUNK)], sem_out).wait()


_sc_update = pl.kernel(
    _body,
    out_type=(
        jax.ShapeDtypeStruct((MAX_TOKENS,), jnp.int32),
        jax.ShapeDtypeStruct((MAX_TOKENS,), jnp.int32),
        jax.ShapeDtypeStruct((MAX_TOKENS,), jnp.float32),
        jax.ShapeDtypeStruct((FW,), jnp.int32),
    ),
    mesh=_mesh,
    scratch_types=[
        pltpu.VMEM((SCAL,), jnp.int32),
        pltpu.VMEM((CHUNK,), jnp.int32),
        pltpu.VMEM((CHUNK,), jnp.int32),
        pltpu.VMEM((CHUNK,), jnp.float32),
        pltpu.VMEM((NSTAGE,), jnp.int32),
        pltpu.VMEM((NSTAGE,), jnp.int32),
        pltpu.VMEM((NSTAGE,), jnp.float32),
        pltpu.VMEM((2 * 16,), jnp.int32),
        pltpu.SemaphoreType.DMA,
        pltpu.SemaphoreType.DMA,
        pltpu.SemaphoreType.DMA,
        pltpu.SemaphoreType.DMA,
        pltpu.SemaphoreType.DMA,
    ],
    compiler_params=pltpu.CompilerParams(needs_layout_passes=False),
)


def kernel(tokens_buf, slot_ids_buf, logprobs_buf, num_tokens, finished,
           new_tokens, new_slot_ids, new_logprobs, num_new_tokens,
           finished_snapshot):
    t = jnp.asarray(num_tokens, jnp.int32)
    n = jnp.asarray(num_new_tokens, jnp.int32)
    fin_w = lax.bitcast_convert_type(
        finished.astype(jnp.uint8).reshape(FW, 4), jnp.int32)
    snap_w = lax.bitcast_convert_type(
        finished_snapshot.astype(jnp.uint8).reshape(FW, 4), jnp.int32)
    scal = jnp.concatenate(
        [jnp.broadcast_to(t, (16,)), jnp.broadcast_to(n, (16,)),
         fin_w, snap_w])
    out_tok, out_sid, out_lp, out_fin = _sc_update(
        tokens_buf, slot_ids_buf, logprobs_buf, scal,
        new_tokens, new_slot_ids, new_logprobs)
    fin_bool = lax.bitcast_convert_type(out_fin, jnp.uint8).reshape(MAX_SEQS)
    return (out_tok, out_sid, out_lp, t + n, fin_bool.astype(jnp.bool_))
